# baseline (speedup 1.0000x reference)
import jax
import jax.numpy as jnp
from jax import lax
from jax.experimental import pallas as pl
from jax.experimental.pallas import tpu as pltpu
from jax.experimental.pallas import tpu_sc as plsc

VOCAB = 100000
MAX_LEN = 200
EMBED_DIM = 64
BATCH = 1024

NC = 2
NS = 16
NW = NC * NS
NBT = BATCH // 128
NSG = NW // NBT
SPG = MAX_LEN // NSG
LANES = 16
GPB = 128 // LANES
IDX_ROWS = SPG // 8 + 2


def _body(x_hbm, tab_hbm, pos_hbm, out_hbm, idx_v, pos_v, rows_v, outb_v,
          gsem0, gsem1, ssem0, ssem1):
    wid = lax.axis_index("s") * NC + lax.axis_index("c")
    bt = wid % NBT
    sgrp = wid // NBT
    p0 = sgrp * SPG
    st0 = p0 // 8
    gsem = (gsem0, gsem1)
    ssem = (ssem0, ssem1)

    pltpu.sync_copy(x_hbm.at[pl.ds(st0, IDX_ROWS), pl.ds(bt, 1)], idx_v)
    pltpu.sync_copy(pos_hbm.at[pl.ds(p0, SPG)], pos_v.at[:, pl.ds(0, EMBED_DIM)])

    def idx_ref(u):
        off = u + p0 % 8
        return idx_v.at[off // 8, 0, off % 8]

    def issue_gather(u, b):
        pltpu.async_copy(tab_hbm.at[idx_ref(u)], rows_v.at[b], gsem[b])

    def drain_gather(b):
        pltpu.make_async_copy(
            tab_hbm.at[idx_ref(0)], rows_v.at[b], gsem[b]).wait()

    def issue_scatter(u, b):
        pltpu.async_copy(
            outb_v.at[b], out_hbm.at[p0 + u, :, pl.ds(bt, 1)], ssem[b])

    def drain_scatter(b):
        pltpu.make_async_copy(
            outb_v.at[b], out_hbm.at[0, :, pl.ds(0, 1)], ssem[b]).wait()

    rowvecs = [lax.iota(jnp.int32, LANES) + g * LANES for g in range(GPB)]

    def compute(u, b):
        @plsc.parallel_loop(0, 8, 1)
        def _(ct):
            pv = pos_v[u, pl.ds(ct * 8, LANES)]
            for ci in range(8):
                c = ct * 8 + ci
                p = pv[ci]
                col = jnp.zeros((LANES,), jnp.int32) + c
                for g in range(GPB):
                    val = plsc.load_gather(rows_v.at[b], [rowvecs[g], col])
                    outb_v[b, ct, 0, ci, pl.ds(g * LANES, LANES)] = val + p

    issue_gather(0, 0)

    def step(i, _):
        u0 = 2 * i

        @pl.when(i > 0)
        def _():
            drain_scatter(1)

        issue_gather(u0 + 1, 1)

        drain_gather(0)
        compute(u0, 0)
        issue_scatter(u0, 0)

        @pl.when(i < SPG // 2 - 1)
        def _():
            drain_scatter(0)
            issue_gather(u0 + 2, 0)

        drain_gather(1)
        compute(u0 + 1, 1)
        issue_scatter(u0 + 1, 1)
        return 0

    lax.fori_loop(0, SPG // 2, step, 0)
    drain_scatter(0)
    drain_scatter(1)


@jax.jit
def kernel(x, token_table, pos_table):
    x6 = (x.astype(jnp.int32).T
          .reshape(MAX_LEN // 8, 8, NBT, 128).transpose(0, 2, 1, 3))
    mesh = plsc.VectorSubcoreMesh(core_axis_name="c", subcore_axis_name="s")
    out6 = pl.kernel(
        _body,
        out_type=jax.ShapeDtypeStruct(
            (MAX_LEN, 8, NBT, 8, 128), jnp.float32),
        mesh=mesh,
        scratch_types=[
            pltpu.VMEM((IDX_ROWS, 1, 8, 128), jnp.int32),
            pltpu.VMEM((SPG, 80), jnp.float32),
            pltpu.VMEM((2, 128, EMBED_DIM), jnp.float32),
            pltpu.VMEM((2, 8, 1, 8, 128), jnp.float32),
            pltpu.SemaphoreType.DMA,
            pltpu.SemaphoreType.DMA,
            pltpu.SemaphoreType.DMA,
            pltpu.SemaphoreType.DMA,
        ],
        compiler_params=pltpu.CompilerParams(
            use_tc_tiling_on_sc=False, needs_layout_passes=False),
    )(x6, token_table, pos_table)
    return out6.transpose(2, 4, 0, 1, 3).reshape(BATCH, MAX_LEN, EMBED_DIM)


# speedup vs baseline: 1.3032x; 1.3032x over previous
import jax
import jax.numpy as jnp
from jax import lax
from jax.experimental import pallas as pl
from jax.experimental.pallas import tpu as pltpu
from jax.experimental.pallas import tpu_sc as plsc

VOCAB = 100000
MAX_LEN = 200
EMBED_DIM = 64
BATCH = 1024

NC = 2
NS = 16
NW = NC * NS
NBT = BATCH // 128
NSG = NW // NBT
SPG = MAX_LEN // NSG
LANES = 16
GPB = 128 // LANES
IDX_ROWS = SPG // 8 + 2
ROW_PITCH = 76


def _body(x_hbm, tab_hbm, pos_hbm, out_hbm, idx_v, pos_v, rows_v, skew_v, outb_v,
          gsem0, gsem1, ssem0, ssem1):
    wid = lax.axis_index("s") * NC + lax.axis_index("c")
    bt = wid % NBT
    sgrp = wid // NBT
    p0 = sgrp * SPG
    st0 = p0 // 8
    gsem = (gsem0, gsem1)
    ssem = (ssem0, ssem1)

    pltpu.sync_copy(x_hbm.at[pl.ds(st0, IDX_ROWS), pl.ds(bt, 1)], idx_v)
    pltpu.sync_copy(pos_hbm.at[pl.ds(p0, SPG)], pos_v.at[:, pl.ds(0, EMBED_DIM)])

    def idx_ref(u):
        off = u + p0 % 8
        return idx_v.at[off // 8, 0, off % 8]

    def issue_gather(u, b):
        pltpu.async_copy(tab_hbm.at[idx_ref(u)], rows_v.at[b], gsem[b])

    def drain_gather(b):
        pltpu.make_async_copy(
            tab_hbm.at[idx_ref(0)], rows_v.at[b], gsem[b]).wait()

    def issue_scatter(u, b):
        pltpu.async_copy(
            outb_v.at[b], out_hbm.at[p0 + u, :, pl.ds(bt, 1)], ssem[b])

    def drain_scatter(b):
        pltpu.make_async_copy(
            outb_v.at[b], out_hbm.at[0, :, pl.ds(0, 1)], ssem[b]).wait()

    rowvecs = [lax.iota(jnp.int32, LANES) + g * LANES for g in range(GPB)]

    def compute(u, b):
        pvs = [pos_v[u, pl.ds(k * LANES, LANES)] for k in range(4)]

        @plsc.parallel_loop(0, 128, 1, unroll=4)
        def _(r):
            for k in range(4):
                sl = pl.ds(k * LANES, LANES)
                skew_v[r, sl] = rows_v[b, r, sl] + pvs[k]

        @plsc.parallel_loop(0, 8, 1)
        def _(ct):
            for ci in range(8):
                c = ct * 8 + ci
                col = jnp.zeros((LANES,), jnp.int32) + c
                vals = [plsc.load_gather(skew_v, [rowvecs[g], col])
                        for g in range(GPB)]
                for g in range(GPB):
                    outb_v[b, ct, 0, ci, pl.ds(g * LANES, LANES)] = vals[g]

    issue_gather(0, 0)

    def step(i, _):
        u0 = 2 * i

        @pl.when(i > 0)
        def _():
            drain_scatter(1)

        issue_gather(u0 + 1, 1)

        drain_gather(0)
        compute(u0, 0)
        issue_scatter(u0, 0)

        @pl.when(i < SPG // 2 - 1)
        def _():
            drain_scatter(0)
            issue_gather(u0 + 2, 0)

        drain_gather(1)
        compute(u0 + 1, 1)
        issue_scatter(u0 + 1, 1)
        return 0

    lax.fori_loop(0, SPG // 2, step, 0)
    drain_scatter(0)
    drain_scatter(1)


@jax.jit
def kernel(x, token_table, pos_table):
    x6 = (x.astype(jnp.int32).T
          .reshape(MAX_LEN // 8, 8, NBT, 128).transpose(0, 2, 1, 3))
    mesh = plsc.VectorSubcoreMesh(core_axis_name="c", subcore_axis_name="s")
    out6 = pl.kernel(
        _body,
        out_type=jax.ShapeDtypeStruct(
            (MAX_LEN, 8, NBT, 8, 128), jnp.float32),
        mesh=mesh,
        scratch_types=[
            pltpu.VMEM((IDX_ROWS, 1, 8, 128), jnp.int32),
            pltpu.VMEM((SPG, 80), jnp.float32),
            pltpu.VMEM((2, 128, EMBED_DIM), jnp.float32),
            pltpu.VMEM((128, ROW_PITCH), jnp.float32),
            pltpu.VMEM((2, 8, 1, 8, 128), jnp.float32),
            pltpu.SemaphoreType.DMA,
            pltpu.SemaphoreType.DMA,
            pltpu.SemaphoreType.DMA,
            pltpu.SemaphoreType.DMA,
        ],
        compiler_params=pltpu.CompilerParams(
            use_tc_tiling_on_sc=False, needs_layout_passes=False),
    )(x6, token_table, pos_table)
    return out6.transpose(2, 4, 0, 1, 3).reshape(BATCH, MAX_LEN, EMBED_DIM)


# speedup vs baseline: 1.7147x; 1.3157x over previous
import jax
import jax.numpy as jnp
from jax import lax
from jax.experimental import pallas as pl
from jax.experimental.pallas import tpu as pltpu
from jax.experimental.pallas import tpu_sc as plsc

VOCAB = 100000
MAX_LEN = 200
EMBED_DIM = 64
BATCH = 1024

NC = 2
NS = 16
NW = NC * NS
NBT = BATCH // 128
NSG = NW // NBT
SPG = MAX_LEN // NSG
LANES = 16
GPB = 128 // LANES
IDX_ROWS = SPG // 8 + 2
ROW_PITCH = 72


def _body(x_hbm, tab_hbm, pos_hbm, out_hbm, idx_v, pos_v, rows_v, skew_v, outb_v,
          gsem0, gsem1, ssem0, ssem1):
    wid = lax.axis_index("s") * NC + lax.axis_index("c")
    bt = wid % NBT
    sgrp = wid // NBT
    p0 = sgrp * SPG
    st0 = p0 // 8
    gsem = (gsem0, gsem1)
    ssem = (ssem0, ssem1)

    pltpu.sync_copy(x_hbm.at[pl.ds(st0, IDX_ROWS), pl.ds(bt, 1)], idx_v)
    pltpu.sync_copy(pos_hbm.at[pl.ds(p0, SPG)], pos_v.at[:, pl.ds(0, EMBED_DIM)])

    def idx_ref(u):
        off = u + p0 % 8
        return idx_v.at[off // 8, 0, off % 8]

    def issue_gather(u, b):
        pltpu.async_copy(tab_hbm.at[idx_ref(u)], rows_v.at[b], gsem[b])

    def drain_gather(b):
        pltpu.make_async_copy(
            tab_hbm.at[idx_ref(0)], rows_v.at[b], gsem[b]).wait()

    def issue_scatter(u, b):
        pltpu.async_copy(
            outb_v.at[b], out_hbm.at[p0 + u, :, pl.ds(bt, 1)], ssem[b])

    def drain_scatter(b):
        pltpu.make_async_copy(
            outb_v.at[b], out_hbm.at[0, :, pl.ds(0, 1)], ssem[b]).wait()

    rowvecs = [lax.iota(jnp.int32, LANES) + g * LANES for g in range(GPB)]

    def compute(u, b):
        pvs = [pos_v[u, pl.ds(k * LANES, LANES)] for k in range(4)]

        @plsc.parallel_loop(0, 128, 1, unroll=4)
        def _(r):
            for k in range(4):
                sl = pl.ds(k * LANES, LANES)
                skew_v[r, sl] = rows_v[b, r, sl] + pvs[k]

        @plsc.parallel_loop(0, 8, 1)
        def _(ct):
            for ci in range(8):
                c = ct * 8 + ci
                col = jnp.zeros((LANES,), jnp.int32) + c
                vals = [plsc.load_gather(skew_v, [rowvecs[g], col])
                        for g in range(GPB)]
                for g in range(GPB):
                    outb_v[b, ct, 0, ci, pl.ds(g * LANES, LANES)] = vals[g]

    issue_gather(0, 0)

    def step(i, _):
        u0 = 2 * i

        @pl.when(i > 0)
        def _():
            drain_scatter(1)

        issue_gather(u0 + 1, 1)

        drain_gather(0)
        compute(u0, 0)
        issue_scatter(u0, 0)

        @pl.when(i < SPG // 2 - 1)
        def _():
            drain_scatter(0)
            issue_gather(u0 + 2, 0)

        drain_gather(1)
        compute(u0 + 1, 1)
        issue_scatter(u0 + 1, 1)
        return 0

    lax.fori_loop(0, SPG // 2, step, 0)
    drain_scatter(0)
    drain_scatter(1)


@jax.jit
def kernel(x, token_table, pos_table):
    x6 = (x.astype(jnp.int32).T
          .reshape(MAX_LEN // 8, 8, NBT, 128).transpose(0, 2, 1, 3))
    mesh = plsc.VectorSubcoreMesh(core_axis_name="c", subcore_axis_name="s")
    out6 = pl.kernel(
        _body,
        out_type=jax.ShapeDtypeStruct(
            (MAX_LEN, 8, NBT, 8, 128), jnp.float32),
        mesh=mesh,
        scratch_types=[
            pltpu.VMEM((IDX_ROWS, 1, 8, 128), jnp.int32),
            pltpu.VMEM((SPG, 80), jnp.float32),
            pltpu.VMEM((2, 128, EMBED_DIM), jnp.float32),
            pltpu.VMEM((128, ROW_PITCH), jnp.float32),
            pltpu.VMEM((2, 8, 1, 8, 128), jnp.float32),
            pltpu.SemaphoreType.DMA,
            pltpu.SemaphoreType.DMA,
            pltpu.SemaphoreType.DMA,
            pltpu.SemaphoreType.DMA,
        ],
        compiler_params=pltpu.CompilerParams(
            use_tc_tiling_on_sc=False, needs_layout_passes=False),
    )(x6, token_table, pos_table)
    return out6.transpose(2, 4, 0, 1, 3).reshape(BATCH, MAX_LEN, EMBED_DIM)
